# baseline (speedup 1.0000x reference)
import jax
import jax.numpy as jnp
from jax import lax
from jax.experimental import pallas as pl
from jax.experimental.pallas import tpu as pltpu

_B = 32
_L = 5
_HW = 441
_HWP = 448
_C = 64
_SEG = 2205
_SEGP = 2304
_K = 3
_NCH = _SEGP // 128
_GW = 2304
_NEG = -1e9


def _snorm_body(s_ref, o_ref):
    for l in range(_L):
        s = s_ref[l]
        cn = jnp.sum(s * s, axis=0, keepdims=True)
        s = s * jnp.where(cn > 0, 1.0 / jnp.sqrt(cn), 0.0)
        o_ref[l] = s.astype(jnp.bfloat16)


def _body(a_ref, sn_ref, o_ref):
    a = a_ref[0]
    rn = jnp.sum(a * a, axis=1, keepdims=True)
    a = (a * jnp.where(rn > 0, 1.0 / jnp.sqrt(rn), 0.0)).astype(jnp.bfloat16)
    row = lax.broadcasted_iota(jnp.int32, (_HWP, 1), 0)
    rmask = row < _HW
    col = lax.broadcasted_iota(jnp.int32, (_HWP, 128), 1).astype(jnp.bfloat16)
    lanei = lax.broadcasted_iota(jnp.int32, (_HWP, 128), 1)
    lastmask = (lanei + (_NCH - 1) * 128) < _SEG
    ms = []
    for l in range(_L):
        t1 = jnp.full((_HWP, 128), _NEG, jnp.bfloat16)
        t2 = t1
        t3 = t1
        for g in range(_SEGP // _GW):
            xg = jnp.dot(a, sn_ref[l][:, g * _GW:(g + 1) * _GW],
                         preferred_element_type=jnp.float32
                         ).astype(jnp.bfloat16)
            for cc in range(_GW // 128):
                c = g * (_GW // 128) + cc
                xc = lax.slice(xg, (0, cc * 128), (_HWP, (cc + 1) * 128))
                if c == _NCH - 1:
                    xc = jnp.where(lastmask, xc, jnp.bfloat16(_NEG))
                n1 = jnp.maximum(t1, xc)
                r = jnp.minimum(t1, xc)
                n2 = jnp.maximum(t2, r)
                r = jnp.minimum(t2, r)
                t3 = jnp.maximum(t3, r)
                t1, t2 = n1, n2
        for k in range(_K):
            m = jnp.max(t1, axis=1, keepdims=True)
            ms.append(m)
            if k < _K - 1:
                idx = jnp.min(jnp.where(t1 >= m, col, jnp.bfloat16(128.0)),
                              axis=1, keepdims=True)
                hit = col == idx
                t1 = jnp.where(hit, t2, t1)
                t2 = jnp.where(hit, t3, t2)
                if k == 0:
                    t3 = jnp.where(hit, jnp.bfloat16(_NEG), t3)
    M = jnp.concatenate(ms, axis=1).astype(jnp.float32)
    M = jnp.where(rmask, jax.nn.sigmoid(M), 0.0)
    csum = jnp.sum(M, axis=0, keepdims=True)
    for l in range(_L):
        o_ref[0, 0, l] = (csum[0, _K * l] + csum[0, _K * l + 1]
                          + csum[0, _K * l + 2])


def kernel(anchor, support_set):
    a = anchor.reshape(_B, _C, _HW).transpose(0, 2, 1)
    a = jnp.pad(a, ((0, 0), (0, _HWP - _HW), (0, 0)))
    s = support_set.reshape(_L * 5, _C, _HW).transpose(0, 2, 1)
    s = s.reshape(_L, _SEG, _C)
    s = jnp.pad(s, ((0, 0), (0, _SEGP - _SEG), (0, 0)))
    s = s.transpose(0, 2, 1)
    sn = pl.pallas_call(
        _snorm_body,
        out_shape=jax.ShapeDtypeStruct((_L, _C, _SEGP), jnp.bfloat16),
    )(s)
    out = pl.pallas_call(
        _body,
        grid=(_B,),
        in_specs=[
            pl.BlockSpec((1, _HWP, _C), lambda b: (b, 0, 0)),
            pl.BlockSpec((_L, _C, _SEGP), lambda b: (0, 0, 0)),
        ],
        out_specs=pl.BlockSpec((1, 1, _L), lambda b: (b, 0, 0),
                               memory_space=pltpu.SMEM),
        out_shape=jax.ShapeDtypeStruct((_B, 1, _L), jnp.float32),
    )(a, sn)
    return out.reshape(_B, _L)


# speedup vs baseline: 1.0765x; 1.0765x over previous
import jax
import jax.numpy as jnp
from jax import lax
from jax.experimental import pallas as pl
from jax.experimental.pallas import tpu as pltpu

_B = 32
_L = 5
_HW = 441
_HWP = 448
_C = 64
_SEG = 2205
_SEGP = 2304
_K = 3
_NCH = _SEGP // 128
_GW = 2304
_NEG = -1e9
_BB = 2


def _snorm_body(s_ref, o_ref):
    for l in range(_L):
        s = s_ref[l]
        cn = jnp.sum(s * s, axis=0, keepdims=True)
        s = s * jnp.where(cn > 0, 1.0 / jnp.sqrt(cn), 0.0)
        o_ref[l] = s.astype(jnp.bfloat16)


def _body(a_ref, sn_ref, o_ref):
    row = lax.broadcasted_iota(jnp.int32, (_HWP, 1), 0)
    rmask = row < _HW
    col = lax.broadcasted_iota(jnp.int32, (_HWP, 128), 1).astype(jnp.bfloat16)
    lanei = lax.broadcasted_iota(jnp.int32, (_HWP, 128), 1)
    lastmask = (lanei + (_NCH - 1) * 128) < _SEG
    for bi in range(_BB):
        a = a_ref[bi]
        rn = jnp.sum(a * a, axis=1, keepdims=True)
        a = (a * jnp.where(rn > 0, 1.0 / jnp.sqrt(rn), 0.0)
             ).astype(jnp.bfloat16)
        ms = []
        for l in range(_L):
            t1 = jnp.full((_HWP, 128), _NEG, jnp.bfloat16)
            t2 = t1
            t3 = t1
            for g in range(_SEGP // _GW):
                xg = jnp.dot(a, sn_ref[l][:, g * _GW:(g + 1) * _GW],
                             preferred_element_type=jnp.float32
                             ).astype(jnp.bfloat16)
                for cc in range(_GW // 128):
                    c = g * (_GW // 128) + cc
                    xc = lax.slice(xg, (0, cc * 128), (_HWP, (cc + 1) * 128))
                    if c == _NCH - 1:
                        xc = jnp.where(lastmask, xc, jnp.bfloat16(_NEG))
                    n1 = jnp.maximum(t1, xc)
                    r = jnp.minimum(t1, xc)
                    n2 = jnp.maximum(t2, r)
                    r = jnp.minimum(t2, r)
                    t3 = jnp.maximum(t3, r)
                    t1, t2 = n1, n2
            for k in range(_K):
                m = jnp.max(t1, axis=1, keepdims=True)
                ms.append(m)
                if k < _K - 1:
                    idx = jnp.min(jnp.where(t1 >= m, col,
                                            jnp.bfloat16(128.0)),
                                  axis=1, keepdims=True)
                    hit = col == idx
                    t1 = jnp.where(hit, t2, t1)
                    t2 = jnp.where(hit, t3, t2)
                    if k == 0:
                        t3 = jnp.where(hit, jnp.bfloat16(_NEG), t3)
        M = jnp.concatenate(ms, axis=1).astype(jnp.float32)
        M = jnp.where(rmask, jax.nn.sigmoid(M), 0.0)
        csum = jnp.sum(M, axis=0, keepdims=True)
        for l in range(_L):
            o_ref[bi, 0, l] = (csum[0, _K * l] + csum[0, _K * l + 1]
                               + csum[0, _K * l + 2])


def kernel(anchor, support_set):
    a = anchor.reshape(_B, _C, _HW).transpose(0, 2, 1)
    a = jnp.pad(a, ((0, 0), (0, _HWP - _HW), (0, 0)))
    s = support_set.reshape(_L * 5, _C, _HW).transpose(0, 2, 1)
    s = s.reshape(_L, _SEG, _C)
    s = jnp.pad(s, ((0, 0), (0, _SEGP - _SEG), (0, 0)))
    s = s.transpose(0, 2, 1)
    sn = pl.pallas_call(
        _snorm_body,
        out_shape=jax.ShapeDtypeStruct((_L, _C, _SEGP), jnp.bfloat16),
    )(s)
    out = pl.pallas_call(
        _body,
        grid=(_B // _BB,),
        in_specs=[
            pl.BlockSpec((_BB, _HWP, _C), lambda b: (b, 0, 0)),
            pl.BlockSpec((_L, _C, _SEGP), lambda b: (0, 0, 0)),
        ],
        out_specs=pl.BlockSpec((_BB, 1, _L), lambda b: (b, 0, 0),
                               memory_space=pltpu.SMEM),
        out_shape=jax.ShapeDtypeStruct((_B, 1, _L), jnp.float32),
    )(a, sn)
    return out.reshape(_B, _L)


# speedup vs baseline: 1.1143x; 1.0351x over previous
import jax
import jax.numpy as jnp
from jax import lax
from jax.experimental import pallas as pl
from jax.experimental.pallas import tpu as pltpu

_B = 32
_L = 5
_HW = 441
_HWP = 448
_C = 64
_SEG = 2205
_SEGP = 2304
_K = 3
_NCH = _SEGP // 128
_GW = 2304
_NEG = -1e9
_BB = 4


def _snorm_body(s_ref, o_ref):
    for l in range(_L):
        s = s_ref[l]
        cn = jnp.sum(s * s, axis=0, keepdims=True)
        s = s * jnp.where(cn > 0, 1.0 / jnp.sqrt(cn), 0.0)
        o_ref[l] = s.astype(jnp.bfloat16)


def _body(a_ref, sn_ref, o_ref):
    row = lax.broadcasted_iota(jnp.int32, (_HWP, 1), 0)
    rmask = row < _HW
    col = lax.broadcasted_iota(jnp.int32, (_HWP, 128), 1).astype(jnp.bfloat16)
    lanei = lax.broadcasted_iota(jnp.int32, (_HWP, 128), 1)
    lastmask = (lanei + (_NCH - 1) * 128) < _SEG
    for bi in range(_BB):
        a = a_ref[bi]
        rn = jnp.sum(a * a, axis=1, keepdims=True)
        a = (a * jnp.where(rn > 0, 1.0 / jnp.sqrt(rn), 0.0)
             ).astype(jnp.bfloat16)
        ms = []
        for l in range(_L):
            t1 = jnp.full((_HWP, 128), _NEG, jnp.bfloat16)
            t2 = t1
            t3 = t1
            for g in range(_SEGP // _GW):
                xg = jnp.dot(a, sn_ref[l][:, g * _GW:(g + 1) * _GW],
                             preferred_element_type=jnp.float32
                             ).astype(jnp.bfloat16)
                for cc in range(_GW // 128):
                    c = g * (_GW // 128) + cc
                    xc = lax.slice(xg, (0, cc * 128), (_HWP, (cc + 1) * 128))
                    if c == _NCH - 1:
                        xc = jnp.where(lastmask, xc, jnp.bfloat16(_NEG))
                    n1 = jnp.maximum(t1, xc)
                    r = jnp.minimum(t1, xc)
                    n2 = jnp.maximum(t2, r)
                    r = jnp.minimum(t2, r)
                    t3 = jnp.maximum(t3, r)
                    t1, t2 = n1, n2
            for k in range(_K):
                m = jnp.max(t1, axis=1, keepdims=True)
                ms.append(m)
                if k < _K - 1:
                    idx = jnp.min(jnp.where(t1 >= m, col,
                                            jnp.bfloat16(128.0)),
                                  axis=1, keepdims=True)
                    hit = col == idx
                    t1 = jnp.where(hit, t2, t1)
                    t2 = jnp.where(hit, t3, t2)
                    if k == 0:
                        t3 = jnp.where(hit, jnp.bfloat16(_NEG), t3)
        M = jnp.concatenate(ms, axis=1).astype(jnp.float32)
        M = jnp.where(rmask, jax.nn.sigmoid(M), 0.0)
        csum = jnp.sum(M, axis=0, keepdims=True)
        for l in range(_L):
            o_ref[bi, 0, l] = (csum[0, _K * l] + csum[0, _K * l + 1]
                               + csum[0, _K * l + 2])


def kernel(anchor, support_set):
    a = anchor.reshape(_B, _C, _HW).transpose(0, 2, 1)
    a = jnp.pad(a, ((0, 0), (0, _HWP - _HW), (0, 0)))
    s = support_set.reshape(_L * 5, _C, _HW).transpose(0, 2, 1)
    s = s.reshape(_L, _SEG, _C)
    s = jnp.pad(s, ((0, 0), (0, _SEGP - _SEG), (0, 0)))
    s = s.transpose(0, 2, 1)
    sn = pl.pallas_call(
        _snorm_body,
        out_shape=jax.ShapeDtypeStruct((_L, _C, _SEGP), jnp.bfloat16),
    )(s)
    out = pl.pallas_call(
        _body,
        grid=(_B // _BB,),
        in_specs=[
            pl.BlockSpec((_BB, _HWP, _C), lambda b: (b, 0, 0)),
            pl.BlockSpec((_L, _C, _SEGP), lambda b: (0, 0, 0)),
        ],
        out_specs=pl.BlockSpec((_BB, 1, _L), lambda b: (b, 0, 0),
                               memory_space=pltpu.SMEM),
        out_shape=jax.ShapeDtypeStruct((_B, 1, _L), jnp.float32),
    )(a, sn)
    return out.reshape(_B, _L)


# speedup vs baseline: 1.1327x; 1.0165x over previous
import jax
import jax.numpy as jnp
from jax import lax
from jax.experimental import pallas as pl
from jax.experimental.pallas import tpu as pltpu

_B = 32
_L = 5
_HW = 441
_HWP = 448
_C = 64
_SEG = 2205
_SEGP = 2304
_K = 3
_NCH = _SEGP // 128
_GW = 2304
_NEG = -1e9
_BB = 8


def _snorm_body(s_ref, o_ref):
    for l in range(_L):
        s = s_ref[l]
        cn = jnp.sum(s * s, axis=0, keepdims=True)
        s = s * jnp.where(cn > 0, 1.0 / jnp.sqrt(cn), 0.0)
        o_ref[l] = s.astype(jnp.bfloat16)


def _body(a_ref, sn_ref, o_ref):
    row = lax.broadcasted_iota(jnp.int32, (_HWP, 1), 0)
    rmask = row < _HW
    col = lax.broadcasted_iota(jnp.int32, (_HWP, 128), 1).astype(jnp.bfloat16)
    lanei = lax.broadcasted_iota(jnp.int32, (_HWP, 128), 1)
    lastmask = (lanei + (_NCH - 1) * 128) < _SEG
    for bi in range(_BB):
        a = a_ref[bi]
        rn = jnp.sum(a * a, axis=1, keepdims=True)
        a = (a * jnp.where(rn > 0, 1.0 / jnp.sqrt(rn), 0.0)
             ).astype(jnp.bfloat16)
        ms = []
        for l in range(_L):
            t1 = jnp.full((_HWP, 128), _NEG, jnp.bfloat16)
            t2 = t1
            t3 = t1
            for g in range(_SEGP // _GW):
                xg = jnp.dot(a, sn_ref[l][:, g * _GW:(g + 1) * _GW],
                             preferred_element_type=jnp.float32
                             ).astype(jnp.bfloat16)
                for cc in range(_GW // 128):
                    c = g * (_GW // 128) + cc
                    xc = lax.slice(xg, (0, cc * 128), (_HWP, (cc + 1) * 128))
                    if c == _NCH - 1:
                        xc = jnp.where(lastmask, xc, jnp.bfloat16(_NEG))
                    n1 = jnp.maximum(t1, xc)
                    r = jnp.minimum(t1, xc)
                    n2 = jnp.maximum(t2, r)
                    r = jnp.minimum(t2, r)
                    t3 = jnp.maximum(t3, r)
                    t1, t2 = n1, n2
            for k in range(_K):
                m = jnp.max(t1, axis=1, keepdims=True)
                ms.append(m)
                if k < _K - 1:
                    idx = jnp.min(jnp.where(t1 >= m, col,
                                            jnp.bfloat16(128.0)),
                                  axis=1, keepdims=True)
                    hit = col == idx
                    t1 = jnp.where(hit, t2, t1)
                    t2 = jnp.where(hit, t3, t2)
                    if k == 0:
                        t3 = jnp.where(hit, jnp.bfloat16(_NEG), t3)
        M = jnp.concatenate(ms, axis=1).astype(jnp.float32)
        M = jnp.where(rmask, jax.nn.sigmoid(M), 0.0)
        csum = jnp.sum(M, axis=0, keepdims=True)
        for l in range(_L):
            o_ref[bi, 0, l] = (csum[0, _K * l] + csum[0, _K * l + 1]
                               + csum[0, _K * l + 2])


def kernel(anchor, support_set):
    a = anchor.reshape(_B, _C, _HW).transpose(0, 2, 1)
    a = jnp.pad(a, ((0, 0), (0, _HWP - _HW), (0, 0)))
    s = support_set.reshape(_L * 5, _C, _HW).transpose(0, 2, 1)
    s = s.reshape(_L, _SEG, _C)
    s = jnp.pad(s, ((0, 0), (0, _SEGP - _SEG), (0, 0)))
    s = s.transpose(0, 2, 1)
    sn = pl.pallas_call(
        _snorm_body,
        out_shape=jax.ShapeDtypeStruct((_L, _C, _SEGP), jnp.bfloat16),
    )(s)
    out = pl.pallas_call(
        _body,
        grid=(_B // _BB,),
        in_specs=[
            pl.BlockSpec((_BB, _HWP, _C), lambda b: (b, 0, 0)),
            pl.BlockSpec((_L, _C, _SEGP), lambda b: (0, 0, 0)),
        ],
        out_specs=pl.BlockSpec((_BB, 1, _L), lambda b: (b, 0, 0),
                               memory_space=pltpu.SMEM),
        out_shape=jax.ShapeDtypeStruct((_B, 1, _L), jnp.float32),
    )(a, sn)
    return out.reshape(_B, _L)
